# baseline (speedup 1.0000x reference)
import functools

import jax
import jax.numpy as jnp
from jax import lax
from jax.experimental import pallas as pl
from jax.experimental.pallas import tpu as pltpu
from jax.experimental.pallas import tpu_sc as plsc

_NC = 2
_NS = 16
_NW = _NC * _NS


def _sc_gather(table, idx):
    n = idx.shape[0]
    d = table.shape[1]
    chunk = next(
        c for c in range(8, n + 1, 8) if n % c == 0 and n // c <= _NW
    )
    n_active = n // chunk
    mesh = plsc.VectorSubcoreMesh(core_axis_name="c", subcore_axis_name="s")

    @functools.partial(
        pl.kernel,
        mesh=mesh,
        out_type=jax.ShapeDtypeStruct((n, d), jnp.float32),
        scratch_types=[
            pltpu.VMEM((chunk,), jnp.int32),
            pltpu.VMEM((chunk, d), jnp.float32),
            pltpu.SemaphoreType.DMA,
        ],
    )
    def k(table_hbm, idx_hbm, out_hbm, idx_v, rows_v, sem):
        wid = lax.axis_index("s") * _NC + lax.axis_index("c")
        base = wid * chunk

        @pl.when(wid < n_active)
        def _run():
            pltpu.sync_copy(idx_hbm.at[pl.ds(base, chunk)], idx_v)
            pltpu.async_copy(table_hbm.at[idx_v], rows_v, sem).wait()
            pltpu.sync_copy(rows_v, out_hbm.at[pl.ds(base, chunk)])

    return k(table, idx)


def _mm_body(x_ref, z_ref, o_ref):
    o_ref[...] = lax.dot_general(
        x_ref[...],
        z_ref[...],
        dimension_numbers=(((1,), (1,)), ((), ())),
        preferred_element_type=jnp.float32,
    )


def _tc_matmul(x, z):
    b = x.shape[0]
    n = z.shape[0]
    return pl.pallas_call(
        _mm_body,
        out_shape=jax.ShapeDtypeStruct((b, n), jnp.float32),
    )(x, z)


def kernel(x_data, encoded_labels, emb_table):
    d = emb_table.shape[1]
    idx = encoded_labels.astype(jnp.int32)
    dpad = (-d) % 128
    table = jnp.pad(emb_table, ((0, 0), (0, dpad))) if dpad else emb_table
    x128 = jnp.pad(x_data, ((0, 0), (0, dpad))) if dpad else x_data
    z_label = _sc_gather(table, idx)
    return _tc_matmul(x128, z_label)


# speedup vs baseline: 1.0043x; 1.0043x over previous
import functools

import jax
import jax.numpy as jnp
from jax import lax
from jax.experimental import pallas as pl
from jax.experimental.pallas import tpu as pltpu
from jax.experimental.pallas import tpu_sc as plsc

_NC = 2
_NS = 16
_NW = _NC * _NS


def _sc_gather(table, idx):
    n = idx.shape[0]
    d = table.shape[1]
    chunk = next(
        c for c in range(8, n + 1, 8) if n % c == 0 and n // c <= _NW
    )
    n_active = n // chunk
    mesh = plsc.VectorSubcoreMesh(core_axis_name="c", subcore_axis_name="s")

    c0 = chunk // 2 // 8 * 8
    c1 = chunk - c0
    scratch = {
        "idx_v": pltpu.VMEM((chunk,), jnp.int32),
        "rows_a": pltpu.VMEM((c0, d), jnp.float32),
        "rows_b": pltpu.VMEM((c1, d), jnp.float32),
        "sem_a": pltpu.SemaphoreType.DMA,
        "sem_b": pltpu.SemaphoreType.DMA,
        "sem_oa": pltpu.SemaphoreType.DMA,
        "sem_ob": pltpu.SemaphoreType.DMA,
    }

    @functools.partial(
        pl.kernel,
        mesh=mesh,
        out_type=jax.ShapeDtypeStruct((n, d), jnp.float32),
        scratch_types=scratch,
    )
    def k(table_hbm, idx_hbm, out_hbm, idx_v, rows_a, rows_b, sem_a, sem_b,
          sem_oa, sem_ob):
        wid = lax.axis_index("s") * _NC + lax.axis_index("c")
        base = wid * chunk

        @pl.when(wid < n_active)
        def _run():
            pltpu.sync_copy(idx_hbm.at[pl.ds(base, chunk)], idx_v)
            ga = pltpu.async_copy(table_hbm.at[idx_v.at[pl.ds(0, c0)]],
                                  rows_a, sem_a)
            gb = pltpu.async_copy(table_hbm.at[idx_v.at[pl.ds(c0, c1)]],
                                  rows_b, sem_b)
            ga.wait()
            oa = pltpu.async_copy(rows_a, out_hbm.at[pl.ds(base, c0)], sem_oa)
            gb.wait()
            ob = pltpu.async_copy(rows_b, out_hbm.at[pl.ds(base + c0, c1)],
                                  sem_ob)
            oa.wait()
            ob.wait()

    return k(table, idx)


def _mm_body(d, x_ref, z_ref, o_ref):
    o_ref[...] = lax.dot_general(
        x_ref[...],
        z_ref[:, :d],
        dimension_numbers=(((1,), (1,)), ((), ())),
        preferred_element_type=jnp.float32,
    )


def _tc_matmul(x, z):
    b, d = x.shape
    n = z.shape[0]
    return pl.pallas_call(
        functools.partial(_mm_body, d),
        out_shape=jax.ShapeDtypeStruct((b, n), jnp.float32),
    )(x, z)


def kernel(x_data, encoded_labels, emb_table):
    d = emb_table.shape[1]
    idx = encoded_labels.astype(jnp.int32)
    dpad = (-d) % 128
    table = jnp.pad(emb_table, ((0, 0), (0, dpad))) if dpad else emb_table
    z_label = _sc_gather(table, idx)
    return _tc_matmul(x_data, z_label)


# speedup vs baseline: 1.0075x; 1.0032x over previous
import functools

import jax
import jax.numpy as jnp
from jax import lax
from jax.experimental import pallas as pl
from jax.experimental.pallas import tpu as pltpu
from jax.experimental.pallas import tpu_sc as plsc

_NC = 2
_NS = 16
_NW = _NC * _NS


def _sc_gather(table, idx):
    n = idx.shape[0]
    d = table.shape[1]
    chunk = next(
        c for c in range(8, n + 1, 8) if n % c == 0 and n // c <= _NW
    )
    n_active = n // chunk
    mesh = plsc.VectorSubcoreMesh(core_axis_name="c", subcore_axis_name="s")

    @functools.partial(
        pl.kernel,
        mesh=mesh,
        out_type=jax.ShapeDtypeStruct((n, d), jnp.float32),
        scratch_types=[
            pltpu.VMEM((chunk,), jnp.int32),
            pltpu.VMEM((chunk, d), jnp.float32),
            pltpu.SemaphoreType.DMA,
        ],
    )
    def k(table_hbm, idx_hbm, out_hbm, idx_v, rows_v, sem):
        wid = lax.axis_index("s") * _NC + lax.axis_index("c")
        base = wid * chunk

        @pl.when(wid < n_active)
        def _run():
            pltpu.sync_copy(idx_hbm.at[pl.ds(base, chunk)], idx_v)
            pltpu.async_copy(table_hbm.at[idx_v], rows_v, sem).wait()
            pltpu.sync_copy(rows_v, out_hbm.at[pl.ds(base, chunk)])

    return k(table, idx)


def _mm_body(d, x_ref, z_ref, o_ref):
    o_ref[...] = lax.dot_general(
        x_ref[...],
        z_ref[:, :d],
        dimension_numbers=(((1,), (1,)), ((), ())),
        preferred_element_type=jnp.float32,
    )


def _tc_matmul(x, z):
    b, d = x.shape
    n = z.shape[0]
    return pl.pallas_call(
        functools.partial(_mm_body, d),
        out_shape=jax.ShapeDtypeStruct((b, n), jnp.float32),
    )(x, z)


def kernel(x_data, encoded_labels, emb_table):
    d = emb_table.shape[1]
    idx = encoded_labels.astype(jnp.int32)
    dpad = (-d) % 128
    table = jnp.pad(emb_table, ((0, 0), (0, dpad))) if dpad else emb_table
    z_label = _sc_gather(table, idx)
    return _tc_matmul(x_data, z_label)
